# baseline (speedup 1.0000x reference)
import functools

import jax
import jax.numpy as jnp
from jax import lax
from jax.experimental import pallas as pl
from jax.experimental.pallas import tpu as pltpu
from jax.experimental.pallas import tpu_sc as plsc

N = 10000
D = 128
NC = 2
NS = 16
NW = NC * NS
CHUNK = 128
N_PAD = 10240
ROWS_PER_TILE = N_PAD // NS


_K = 2
_SEGS = 4


@functools.lru_cache(maxsize=None)
def _make_scatter(e_pad):
  per_tile = e_pad // NW
  n_chunks = per_tile // CHUNK
  assert n_chunks % (_K * _SEGS) == 0
  seg_chunks = n_chunks // _SEGS
  seg_edges = seg_chunks * CHUNK
  seg_laps = seg_chunks // _K
  mesh = plsc.VectorSubcoreMesh(
      core_axis_name="c", subcore_axis_name="s", num_cores=NC, num_subcores=NS
  )

  @functools.partial(
      pl.kernel,
      out_type=jax.ShapeDtypeStruct((NC, N_PAD, D), jnp.float32),
      mesh=mesh,
      scratch_types=[
          [pltpu.VMEM((seg_edges,), jnp.int32)] * 2,
          [pltpu.VMEM((seg_edges,), jnp.int32)] * 2,
          pltpu.VMEM((_K, CHUNK, D), jnp.float32),
          pltpu.VMEM_SHARED((N_PAD, D), jnp.float32),
          [pltpu.SemaphoreType.DMA] * _K,
          [pltpu.SemaphoreType.DMA] * 2,
      ],
  )
  def scatter_kernel(g_hbm, src_hbm, dst_hbm, z_hbm, out_hbm,
                     src_v, dst_v, rows_v, acc_sh, gsems, isems):
    c = lax.axis_index("c")
    s = lax.axis_index("s")
    wid = s * NC + c
    base0 = wid * per_tile

    def idx_hbm(hbm, seg):
      return hbm.at[pl.ds(base0 + seg * seg_edges, seg_edges)]

    def sidx(ref, j):
      return ref.at[pl.ds(j * CHUNK, CHUNK)]

    pltpu.async_copy(idx_hbm(src_hbm, 0), src_v[0], isems[0])
    pltpu.async_copy(idx_hbm(dst_hbm, 0), dst_v[0], isems[0])
    pltpu.sync_copy(z_hbm, acc_sh.at[pl.ds(s * ROWS_PER_TILE, ROWS_PER_TILE)])
    pltpu.make_async_copy(idx_hbm(src_hbm, 0), src_v[0], isems[0]).wait()
    pltpu.make_async_copy(idx_hbm(dst_hbm, 0), dst_v[0], isems[0]).wait()
    plsc.subcore_barrier()

    for seg in range(_SEGS):
      pp = seg % 2
      sv, dv = src_v[pp], dst_v[pp]
      if seg > 0:
        pltpu.make_async_copy(idx_hbm(src_hbm, seg), sv, isems[pp]).wait()
        pltpu.make_async_copy(idx_hbm(dst_hbm, seg), dv, isems[pp]).wait()
      if seg < _SEGS - 1:
        pltpu.async_copy(idx_hbm(src_hbm, seg + 1), src_v[1 - pp],
                         isems[1 - pp])
        pltpu.async_copy(idx_hbm(dst_hbm, seg + 1), dst_v[1 - pp],
                         isems[1 - pp])

      for b in range(_K):
        pltpu.async_copy(g_hbm.at[sidx(sv, b)], rows_v.at[b], gsems[b])

      def lap(i, carry, sv=sv, dv=dv):
        for b in range(_K):
          j = i * _K + b
          pltpu.make_async_copy(g_hbm.at[sidx(sv, j)], rows_v.at[b],
                                gsems[b]).wait()
          pltpu.sync_copy(rows_v.at[b], acc_sh.at[sidx(dv, j)], add=True)

          @pl.when(i < seg_laps - 1)
          def _(b=b):
            pltpu.async_copy(g_hbm.at[sidx(sv, (i + 1) * _K + b)],
                             rows_v.at[b], gsems[b])

        return carry

      lax.fori_loop(0, seg_laps, lap, 0)

    plsc.subcore_barrier()

    pltpu.sync_copy(
        acc_sh.at[pl.ds(s * ROWS_PER_TILE, ROWS_PER_TILE)],
        out_hbm.at[c, pl.ds(s * ROWS_PER_TILE, ROWS_PER_TILE)],
    )

  return scatter_kernel


_BM = 1024


def _mm0_body(x_ref, w_ref, o_ref):
  o_ref[...] = jnp.dot(x_ref[...], w_ref[...], preferred_element_type=jnp.float32)


def _mm0(x, w):
  grid = x.shape[0] // _BM
  return pl.pallas_call(
      _mm0_body,
      grid=(grid,),
      in_specs=[
          pl.BlockSpec((_BM, D), lambda i: (i, 0)),
          pl.BlockSpec((D, D), lambda i: (0, 0)),
      ],
      out_specs=pl.BlockSpec((_BM, D), lambda i: (i, 0)),
      out_shape=jax.ShapeDtypeStruct((x.shape[0], D), jnp.float32),
  )(x, w)


def _mm_mid_body(p0_ref, p1_ref, b_ref, w_ref, o_ref):
  h = jnp.maximum(p0_ref[...] + p1_ref[...] + b_ref[...], 0.0)
  o_ref[...] = jnp.dot(h, w_ref[...], preferred_element_type=jnp.float32)


def _mm_mid(p0, p1, b, w):
  grid = p0.shape[0] // _BM
  return pl.pallas_call(
      _mm_mid_body,
      grid=(grid,),
      in_specs=[
          pl.BlockSpec((_BM, D), lambda i: (i, 0)),
          pl.BlockSpec((_BM, D), lambda i: (i, 0)),
          pl.BlockSpec((1, D), lambda i: (0, 0)),
          pl.BlockSpec((D, D), lambda i: (0, 0)),
      ],
      out_specs=pl.BlockSpec((_BM, D), lambda i: (i, 0)),
      out_shape=jax.ShapeDtypeStruct((p0.shape[0], D), jnp.float32),
  )(p0, p1, b, w)


_BF = 1000


def _fin_body(p0_ref, p1_ref, b_ref, o_ref):
  h = p0_ref[...] + p1_ref[...] + b_ref[...]
  nrm = jnp.sqrt(jnp.sum(h * h, axis=1, keepdims=True))
  o_ref[...] = h / jnp.maximum(nrm, 1e-12)


def _fin(p0, p1, b):
  return pl.pallas_call(
      _fin_body,
      grid=(N // _BF,),
      in_specs=[
          pl.BlockSpec((_BF, D), lambda i: (i, 0)),
          pl.BlockSpec((_BF, D), lambda i: (i, 0)),
          pl.BlockSpec((1, D), lambda i: (0, 0)),
      ],
      out_specs=pl.BlockSpec((_BF, D), lambda i: (i, 0)),
      out_shape=jax.ShapeDtypeStruct((N, D), jnp.float32),
  )(p0, p1, b)


def kernel(x, adj, W1, b1, W2, b2, W3, b3):
  e = adj.shape[1]
  gran = NW * CHUNK * _K * _SEGS
  e_pad = ((e + gran - 1) // gran) * gran
  pad = e_pad - e

  src = jnp.concatenate([adj[0], jnp.zeros((pad,), jnp.int32)])
  dst = jnp.concatenate(
      [adj[1], N + (jnp.arange(pad, dtype=jnp.int32) % (N_PAD - N))]
  )
  xp = jnp.concatenate([x, jnp.zeros((N_PAD - N, D), jnp.float32)])
  zeros = jnp.zeros((ROWS_PER_TILE, D), jnp.float32)

  scatter = _make_scatter(e_pad)

  g = _mm0(xp, W1)
  p = scatter(g, src, dst, zeros)
  g = _mm_mid(p[0], p[1], b1.reshape(1, D), W2)
  p = scatter(g, src, dst, zeros)
  g = _mm_mid(p[0], p[1], b2.reshape(1, D), W3)
  p = scatter(g, src, dst, zeros)
  return _fin(p[0], p[1], b3.reshape(1, D))


# speedup vs baseline: 1.2666x; 1.2666x over previous
import functools

import jax
import jax.numpy as jnp
import numpy as np
from jax import lax
from jax.experimental import pallas as pl
from jax.experimental.pallas import tpu as pltpu
from jax.experimental.pallas import tpu_sc as plsc

N = 10000
D = 128
DW = D // 2
NC = 2
NS = 16
NW = NC * NS
CHUNK = 128
N_PAD = 10240
ROWS_PER_TILE = N_PAD // NS

_PERM = np.empty((D,), dtype=np.int32)
for _g in range(D // 32):
  _PERM[32 * _g + 0:32 * _g + 32:2] = 32 * _g + np.arange(16)
  _PERM[32 * _g + 1:32 * _g + 32:2] = 32 * _g + 16 + np.arange(16)

_K = 2
_SEGS = 4


@functools.lru_cache(maxsize=None)
def _make_scatter(e_pad):
  per_tile = e_pad // NW
  n_chunks = per_tile // CHUNK
  assert n_chunks % (_K * _SEGS) == 0
  seg_chunks = n_chunks // _SEGS
  seg_edges = seg_chunks * CHUNK
  seg_laps = seg_chunks // _K
  mesh = plsc.VectorSubcoreMesh(
      core_axis_name="c", subcore_axis_name="s", num_cores=NC, num_subcores=NS
  )

  @functools.partial(
      pl.kernel,
      out_type=jax.ShapeDtypeStruct((NC, N_PAD, D), jnp.float32),
      mesh=mesh,
      compiler_params=pltpu.CompilerParams(use_tc_tiling_on_sc=False),
      scratch_types=[
          [pltpu.VMEM((seg_edges,), jnp.int32)] * 2,
          [pltpu.VMEM((seg_edges,), jnp.int32)] * 2,
          [pltpu.VMEM((CHUNK, DW), jnp.int32)] * _K,
          pltpu.VMEM((CHUNK, D), jnp.float32),
          pltpu.VMEM_SHARED((N_PAD, D), jnp.float32),
          [pltpu.SemaphoreType.DMA] * _K,
          [pltpu.SemaphoreType.DMA] * 2,
      ],
  )
  def scatter_kernel(g_hbm, src_hbm, dst_hbm, z_hbm, out_hbm,
                     src_v, dst_v, rows_v, rowsf_v, acc_sh, gsems, isems):
    c = lax.axis_index("c")
    s = lax.axis_index("s")
    wid = s * NC + c
    base0 = wid * per_tile

    def idx_hbm(hbm, seg):
      return hbm.at[pl.ds(base0 + seg * seg_edges, seg_edges)]

    def sidx(ref, j):
      return ref.at[pl.ds(j * CHUNK, CHUNK)]

    def fire_gather(sv, j, b):
      pltpu.async_copy(g_hbm.at[sidx(sv, j)], rows_v[b], gsems[b])

    def wait_gather(sv, j, b):
      pltpu.make_async_copy(g_hbm.at[sidx(sv, j)], rows_v[b],
                            gsems[b]).wait()

    def unpack_chunk(b):
      def row(r, carry):
        for grp in range(D // 32):
          w = rows_v[b][r, pl.ds(16 * grp, 16)]
          hi = lax.bitcast_convert_type(
              lax.bitwise_and(w, jnp.int32(-65536)), jnp.float32)
          lo = lax.bitcast_convert_type(lax.shift_left(w, 16), jnp.float32)
          rowsf_v[r, pl.ds(32 * grp, 16)] = lo
          rowsf_v[r, pl.ds(32 * grp + 16, 16)] = hi
        return carry

      lax.fori_loop(0, CHUNK, row, 0)

    pltpu.async_copy(idx_hbm(src_hbm, 0), src_v[0], isems[0])
    pltpu.async_copy(idx_hbm(dst_hbm, 0), dst_v[0], isems[0])
    pltpu.sync_copy(z_hbm, acc_sh.at[pl.ds(s * ROWS_PER_TILE, ROWS_PER_TILE)])
    pltpu.make_async_copy(idx_hbm(src_hbm, 0), src_v[0], isems[0]).wait()
    pltpu.make_async_copy(idx_hbm(dst_hbm, 0), dst_v[0], isems[0]).wait()
    plsc.subcore_barrier()

    for seg in range(_SEGS):
      pp = seg % 2
      sv, dv = src_v[pp], dst_v[pp]
      if seg > 0:
        pltpu.make_async_copy(idx_hbm(src_hbm, seg), sv, isems[pp]).wait()
        pltpu.make_async_copy(idx_hbm(dst_hbm, seg), dv, isems[pp]).wait()
      if seg < _SEGS - 1:
        pltpu.async_copy(idx_hbm(src_hbm, seg + 1), src_v[1 - pp],
                         isems[1 - pp])
        pltpu.async_copy(idx_hbm(dst_hbm, seg + 1), dst_v[1 - pp],
                         isems[1 - pp])

      for b in range(_K):
        fire_gather(sv, b, b)

      def lap(i, carry, sv=sv, dv=dv):
        for b in range(_K):
          j = i * _K + b
          wait_gather(sv, j, b)
          unpack_chunk(b)

          @pl.when(j + _K < seg_chunks)
          def _(b=b, j=j, sv=sv):
            fire_gather(sv, j + _K, b)

          pltpu.sync_copy(rowsf_v, acc_sh.at[sidx(dv, j)], add=True)
        return carry

      lax.fori_loop(0, seg_laps, lap, 0)

    plsc.subcore_barrier()

    pltpu.sync_copy(
        acc_sh.at[pl.ds(s * ROWS_PER_TILE, ROWS_PER_TILE)],
        out_hbm.at[c, pl.ds(s * ROWS_PER_TILE, ROWS_PER_TILE)],
    )

  return scatter_kernel


_BM = 1024


def _mm0_body(x_ref, w_ref, o_ref):
  o_ref[...] = jnp.dot(
      x_ref[...], w_ref[...], preferred_element_type=jnp.float32
  ).astype(jnp.bfloat16)


def _mm0(x, w):
  grid = x.shape[0] // _BM
  return pl.pallas_call(
      _mm0_body,
      grid=(grid,),
      in_specs=[
          pl.BlockSpec((_BM, D), lambda i: (i, 0)),
          pl.BlockSpec((D, D), lambda i: (0, 0)),
      ],
      out_specs=pl.BlockSpec((_BM, D), lambda i: (i, 0)),
      out_shape=jax.ShapeDtypeStruct((x.shape[0], D), jnp.bfloat16),
  )(x, w)


def _mm_mid_body(p0_ref, p1_ref, b_ref, w_ref, o_ref):
  h = jnp.maximum(p0_ref[...] + p1_ref[...] + b_ref[...], 0.0)
  o_ref[...] = jnp.dot(
      h, w_ref[...], preferred_element_type=jnp.float32
  ).astype(jnp.bfloat16)


def _mm_mid(p0, p1, b, w):
  grid = p0.shape[0] // _BM
  return pl.pallas_call(
      _mm_mid_body,
      grid=(grid,),
      in_specs=[
          pl.BlockSpec((_BM, D), lambda i: (i, 0)),
          pl.BlockSpec((_BM, D), lambda i: (i, 0)),
          pl.BlockSpec((1, D), lambda i: (0, 0)),
          pl.BlockSpec((D, D), lambda i: (0, 0)),
      ],
      out_specs=pl.BlockSpec((_BM, D), lambda i: (i, 0)),
      out_shape=jax.ShapeDtypeStruct((p0.shape[0], D), jnp.bfloat16),
  )(p0, p1, b, w)


_BF = 1000


def _fin_body(p0_ref, p1_ref, b_ref, o_ref):
  h = p0_ref[...] + p1_ref[...] + b_ref[...]
  nrm = jnp.sqrt(jnp.sum(h * h, axis=1, keepdims=True))
  o_ref[...] = h / jnp.maximum(nrm, 1e-12)


def _fin(p0, p1, b):
  return pl.pallas_call(
      _fin_body,
      grid=(N // _BF,),
      in_specs=[
          pl.BlockSpec((_BF, D), lambda i: (i, 0)),
          pl.BlockSpec((_BF, D), lambda i: (i, 0)),
          pl.BlockSpec((1, D), lambda i: (0, 0)),
      ],
      out_specs=pl.BlockSpec((_BF, D), lambda i: (i, 0)),
      out_shape=jax.ShapeDtypeStruct((N, D), jnp.float32),
  )(p0, p1, b)


def _pack(g_bf):
  return lax.bitcast_convert_type(
      g_bf.reshape(N_PAD, DW, 2), jnp.int32
  )


def kernel(x, adj, W1, b1, W2, b2, W3, b3):
  e = adj.shape[1]
  gran = NW * CHUNK * _K * _SEGS
  e_pad = ((e + gran - 1) // gran) * gran
  pad = e_pad - e

  src = jnp.concatenate([adj[0], jnp.zeros((pad,), jnp.int32)])
  dst = jnp.concatenate(
      [adj[1], N + (jnp.arange(pad, dtype=jnp.int32) % (N_PAD - N))]
  )
  xp = jnp.concatenate([x, jnp.zeros((N_PAD - N, D), jnp.float32)])
  zeros = jnp.zeros((ROWS_PER_TILE, D), jnp.float32)
  perm = jnp.asarray(_PERM)

  scatter = _make_scatter(e_pad)

  g = _mm0(xp, W1[:, perm])
  p = scatter(_pack(g), src, dst, zeros)
  g = _mm_mid(p[0], p[1], b1.reshape(1, D), W2[:, perm])
  p = scatter(_pack(g), src, dst, zeros)
  g = _mm_mid(p[0], p[1], b2.reshape(1, D), W3[:, perm])
  p = scatter(_pack(g), src, dst, zeros)
  return _fin(p[0], p[1], b3.reshape(1, D))


# speedup vs baseline: 1.3520x; 1.0675x over previous
import functools

import jax
import jax.numpy as jnp
import numpy as np
from jax import lax
from jax.experimental import pallas as pl
from jax.experimental.pallas import tpu as pltpu
from jax.experimental.pallas import tpu_sc as plsc

N = 10000
D = 128
DW = D // 2
NC = 2
NS = 16
NW = NC * NS
CHUNK = 128
N_PAD = 10240
ROWS_PER_TILE = N_PAD // NS

_K = 2
_SEGS = 4


@functools.lru_cache(maxsize=None)
def _make_scatter(e_pad):
  per_tile = e_pad // NW
  n_chunks = per_tile // CHUNK
  assert n_chunks % (_K * _SEGS) == 0
  seg_chunks = n_chunks // _SEGS
  seg_edges = seg_chunks * CHUNK
  seg_laps = seg_chunks // _K
  mesh = plsc.VectorSubcoreMesh(
      core_axis_name="c", subcore_axis_name="s", num_cores=NC, num_subcores=NS
  )

  @functools.partial(
      pl.kernel,
      out_type=jax.ShapeDtypeStruct((NC, N_PAD, D), jnp.float32),
      mesh=mesh,
      compiler_params=pltpu.CompilerParams(use_tc_tiling_on_sc=False),
      scratch_types=[
          [pltpu.VMEM((seg_edges,), jnp.int32)] * 2,
          [pltpu.VMEM((seg_edges,), jnp.int32)] * 2,
          [pltpu.VMEM((CHUNK, DW), jnp.int32)] * _K,
          pltpu.VMEM((CHUNK, D), jnp.float32),
          pltpu.VMEM_SHARED((N_PAD, D), jnp.float32),
          [pltpu.SemaphoreType.DMA] * _K,
          [pltpu.SemaphoreType.DMA] * 2,
      ],
  )
  def scatter_kernel(g_hbm, src_hbm, dst_hbm, z_hbm, out_hbm,
                     src_v, dst_v, rows_v, rowsf_v, acc_sh, gsems, isems):
    c = lax.axis_index("c")
    s = lax.axis_index("s")
    wid = s * NC + c
    base0 = wid * per_tile

    def idx_hbm(hbm, seg):
      return hbm.at[pl.ds(base0 + seg * seg_edges, seg_edges)]

    def sidx(ref, j):
      return ref.at[pl.ds(j * CHUNK, CHUNK)]

    def fire_gather(sv, j, b):
      pltpu.async_copy(g_hbm.at[sidx(sv, j)], rows_v[b], gsems[b])

    def wait_gather(sv, j, b):
      pltpu.make_async_copy(g_hbm.at[sidx(sv, j)], rows_v[b],
                            gsems[b]).wait()

    def unpack_chunk(b):
      def row(r, carry):
        for grp in range(DW // 16):
          w = rows_v[b][r, pl.ds(16 * grp, 16)]
          hi = lax.bitcast_convert_type(
              lax.bitwise_and(w, jnp.int32(-65536)), jnp.float32)
          lo = lax.bitcast_convert_type(lax.shift_left(w, 16), jnp.float32)
          rowsf_v[r, pl.ds(16 * grp, 16)] = lo
          rowsf_v[r, pl.ds(DW + 16 * grp, 16)] = hi
        return carry

      lax.fori_loop(0, CHUNK, row, 0)

    pltpu.async_copy(idx_hbm(src_hbm, 0), src_v[0], isems[0])
    pltpu.async_copy(idx_hbm(dst_hbm, 0), dst_v[0], isems[0])
    pltpu.sync_copy(z_hbm, acc_sh.at[pl.ds(s * ROWS_PER_TILE, ROWS_PER_TILE)])
    pltpu.make_async_copy(idx_hbm(src_hbm, 0), src_v[0], isems[0]).wait()
    pltpu.make_async_copy(idx_hbm(dst_hbm, 0), dst_v[0], isems[0]).wait()
    plsc.subcore_barrier()

    for seg in range(_SEGS):
      pp = seg % 2
      sv, dv = src_v[pp], dst_v[pp]
      if seg > 0:
        pltpu.make_async_copy(idx_hbm(src_hbm, seg), sv, isems[pp]).wait()
        pltpu.make_async_copy(idx_hbm(dst_hbm, seg), dv, isems[pp]).wait()
      if seg < _SEGS - 1:
        pltpu.async_copy(idx_hbm(src_hbm, seg + 1), src_v[1 - pp],
                         isems[1 - pp])
        pltpu.async_copy(idx_hbm(dst_hbm, seg + 1), dst_v[1 - pp],
                         isems[1 - pp])

      for b in range(_K):
        fire_gather(sv, b, b)

      def lap(i, carry, sv=sv, dv=dv):
        for b in range(_K):
          j = i * _K + b
          wait_gather(sv, j, b)
          unpack_chunk(b)

          @pl.when(j + _K < seg_chunks)
          def _(b=b, j=j, sv=sv):
            fire_gather(sv, j + _K, b)

          pltpu.sync_copy(rowsf_v, acc_sh.at[sidx(dv, j)], add=True)
        return carry

      lax.fori_loop(0, seg_laps, lap, 0)

    plsc.subcore_barrier()

    pltpu.sync_copy(
        acc_sh.at[pl.ds(s * ROWS_PER_TILE, ROWS_PER_TILE)],
        out_hbm.at[c, pl.ds(s * ROWS_PER_TILE, ROWS_PER_TILE)],
    )

  return scatter_kernel


_BM = 1024


def _pack_words(d):
  d16 = d.astype(jnp.bfloat16)
  lo = lax.bitcast_convert_type(d16[:, :DW], jnp.uint16).astype(jnp.int32)
  hi = lax.bitcast_convert_type(d16[:, DW:], jnp.uint16).astype(jnp.int32)
  return lax.bitwise_or(lo, lax.shift_left(hi, 16))


def _mm0_body(x_ref, w_ref, o_ref):
  o_ref[...] = _pack_words(jnp.dot(
      x_ref[...], w_ref[...], preferred_element_type=jnp.float32))


def _mm0(x, w):
  grid = x.shape[0] // _BM
  return pl.pallas_call(
      _mm0_body,
      grid=(grid,),
      in_specs=[
          pl.BlockSpec((_BM, D), lambda i: (i, 0)),
          pl.BlockSpec((D, D), lambda i: (0, 0)),
      ],
      out_specs=pl.BlockSpec((_BM, DW), lambda i: (i, 0)),
      out_shape=jax.ShapeDtypeStruct((x.shape[0], DW), jnp.int32),
  )(x, w)


def _mm_mid_body(p0_ref, p1_ref, b_ref, w_ref, o_ref):
  h = jnp.maximum(p0_ref[...] + p1_ref[...] + b_ref[...], 0.0)
  o_ref[...] = _pack_words(jnp.dot(
      h, w_ref[...], preferred_element_type=jnp.float32))


def _mm_mid(p0, p1, b, w):
  grid = p0.shape[0] // _BM
  return pl.pallas_call(
      _mm_mid_body,
      grid=(grid,),
      in_specs=[
          pl.BlockSpec((_BM, D), lambda i: (i, 0)),
          pl.BlockSpec((_BM, D), lambda i: (i, 0)),
          pl.BlockSpec((1, D), lambda i: (0, 0)),
          pl.BlockSpec((D, D), lambda i: (0, 0)),
      ],
      out_specs=pl.BlockSpec((_BM, DW), lambda i: (i, 0)),
      out_shape=jax.ShapeDtypeStruct((p0.shape[0], DW), jnp.int32),
  )(p0, p1, b, w)


_BF = 1000


def _fin_body(p0_ref, p1_ref, b_ref, o_ref):
  h = p0_ref[...] + p1_ref[...] + b_ref[...]
  nrm = jnp.sqrt(jnp.sum(h * h, axis=1, keepdims=True))
  o_ref[...] = h / jnp.maximum(nrm, 1e-12)


def _fin(p0, p1, b):
  return pl.pallas_call(
      _fin_body,
      grid=(N // _BF,),
      in_specs=[
          pl.BlockSpec((_BF, D), lambda i: (i, 0)),
          pl.BlockSpec((_BF, D), lambda i: (i, 0)),
          pl.BlockSpec((1, D), lambda i: (0, 0)),
      ],
      out_specs=pl.BlockSpec((_BF, D), lambda i: (i, 0)),
      out_shape=jax.ShapeDtypeStruct((N, D), jnp.float32),
  )(p0, p1, b)


def kernel(x, adj, W1, b1, W2, b2, W3, b3):
  e = adj.shape[1]
  gran = NW * CHUNK * _K * _SEGS
  e_pad = ((e + gran - 1) // gran) * gran
  pad = e_pad - e

  src = jnp.concatenate([adj[0], jnp.zeros((pad,), jnp.int32)])
  dst = jnp.concatenate(
      [adj[1], N + (jnp.arange(pad, dtype=jnp.int32) % (N_PAD - N))]
  )
  xp = jnp.concatenate([x, jnp.zeros((N_PAD - N, D), jnp.float32)])
  zeros = jnp.zeros((ROWS_PER_TILE, D), jnp.float32)

  scatter = _make_scatter(e_pad)

  g = _mm0(xp, W1)
  p = scatter(g, src, dst, zeros)
  g = _mm_mid(p[0], p[1], b1.reshape(1, D), W2)
  p = scatter(g, src, dst, zeros)
  g = _mm_mid(p[0], p[1], b2.reshape(1, D), W3)
  p = scatter(g, src, dst, zeros)
  return _fin(p[0], p[1], b3.reshape(1, D))


# speedup vs baseline: 1.3532x; 1.0009x over previous
import functools

import jax
import jax.numpy as jnp
import numpy as np
from jax import lax
from jax.experimental import pallas as pl
from jax.experimental.pallas import tpu as pltpu
from jax.experimental.pallas import tpu_sc as plsc

N = 10000
D = 128
DW = D // 2
NC = 2
NS = 16
NW = NC * NS
CHUNK = 128
N_PAD = 10240
ROWS_PER_TILE = N_PAD // NS

_K = 2
_SEGS = 4


@functools.lru_cache(maxsize=None)
def _make_scatter(e_pad):
  per_tile = e_pad // NW
  n_chunks = per_tile // CHUNK
  assert n_chunks % (_K * _SEGS) == 0
  seg_chunks = n_chunks // _SEGS
  seg_edges = seg_chunks * CHUNK
  seg_laps = seg_chunks // _K
  mesh = plsc.VectorSubcoreMesh(
      core_axis_name="c", subcore_axis_name="s", num_cores=NC, num_subcores=NS
  )

  @functools.partial(
      pl.kernel,
      out_type=jax.ShapeDtypeStruct((NC, N_PAD, D), jnp.float32),
      mesh=mesh,
      compiler_params=pltpu.CompilerParams(use_tc_tiling_on_sc=False),
      scratch_types=[
          [pltpu.VMEM((seg_edges,), jnp.int32)] * 2,
          [pltpu.VMEM((seg_edges,), jnp.int32)] * 2,
          [pltpu.VMEM((CHUNK, DW), jnp.int32)] * _K,
          pltpu.VMEM((CHUNK, D), jnp.float32),
          pltpu.VMEM_SHARED((N_PAD, D), jnp.float32),
          [pltpu.SemaphoreType.DMA] * _K,
          [pltpu.SemaphoreType.DMA] * 2,
      ],
  )
  def scatter_kernel(g_hbm, src_hbm, dst_hbm, z_hbm, out_hbm,
                     src_v, dst_v, rows_v, rowsf_v, acc_sh, gsems, isems):
    c = lax.axis_index("c")
    s = lax.axis_index("s")
    wid = s * NC + c
    base0 = wid * per_tile

    def idx_hbm(hbm, seg):
      return hbm.at[pl.ds(base0 + seg * seg_edges, seg_edges)]

    def sidx(ref, j):
      return ref.at[pl.ds(j * CHUNK, CHUNK)]

    def fire_gather(sv, j, b):
      pltpu.async_copy(g_hbm.at[sidx(sv, j)], rows_v[b], gsems[b])

    def wait_gather(sv, j, b):
      pltpu.make_async_copy(g_hbm.at[sidx(sv, j)], rows_v[b],
                            gsems[b]).wait()

    def unpack_chunk(b):
      def row4(i, carry):
        r0 = i * 4
        for dr in range(4):
          r = r0 + dr
          for grp in range(DW // 16):
            w = rows_v[b][r, pl.ds(16 * grp, 16)]
            hi = lax.bitcast_convert_type(
                lax.bitwise_and(w, jnp.int32(-65536)), jnp.float32)
            lo = lax.bitcast_convert_type(lax.shift_left(w, 16), jnp.float32)
            rowsf_v[r, pl.ds(16 * grp, 16)] = lo
            rowsf_v[r, pl.ds(DW + 16 * grp, 16)] = hi
        return carry

      lax.fori_loop(0, CHUNK // 4, row4, 0)

    pltpu.async_copy(idx_hbm(src_hbm, 0), src_v[0], isems[0])
    pltpu.async_copy(idx_hbm(dst_hbm, 0), dst_v[0], isems[0])
    pltpu.sync_copy(z_hbm, acc_sh.at[pl.ds(s * ROWS_PER_TILE, ROWS_PER_TILE)])
    pltpu.make_async_copy(idx_hbm(src_hbm, 0), src_v[0], isems[0]).wait()
    pltpu.make_async_copy(idx_hbm(dst_hbm, 0), dst_v[0], isems[0]).wait()
    plsc.subcore_barrier()

    for seg in range(_SEGS):
      pp = seg % 2
      sv, dv = src_v[pp], dst_v[pp]
      if seg > 0:
        pltpu.make_async_copy(idx_hbm(src_hbm, seg), sv, isems[pp]).wait()
        pltpu.make_async_copy(idx_hbm(dst_hbm, seg), dv, isems[pp]).wait()
      if seg < _SEGS - 1:
        pltpu.async_copy(idx_hbm(src_hbm, seg + 1), src_v[1 - pp],
                         isems[1 - pp])
        pltpu.async_copy(idx_hbm(dst_hbm, seg + 1), dst_v[1 - pp],
                         isems[1 - pp])

      for b in range(_K):
        fire_gather(sv, b, b)

      def lap(i, carry, sv=sv, dv=dv):
        for b in range(_K):
          j = i * _K + b
          wait_gather(sv, j, b)
          unpack_chunk(b)

          @pl.when(j + _K < seg_chunks)
          def _(b=b, j=j, sv=sv):
            fire_gather(sv, j + _K, b)

          pltpu.sync_copy(rowsf_v, acc_sh.at[sidx(dv, j)], add=True)
        return carry

      lax.fori_loop(0, seg_laps, lap, 0)

    plsc.subcore_barrier()

    pltpu.sync_copy(
        acc_sh.at[pl.ds(s * ROWS_PER_TILE, ROWS_PER_TILE)],
        out_hbm.at[c, pl.ds(s * ROWS_PER_TILE, ROWS_PER_TILE)],
    )

  return scatter_kernel


_BM = 1024


def _pack_words(d):
  d16 = d.astype(jnp.bfloat16)
  lo = lax.bitcast_convert_type(d16[:, :DW], jnp.uint16).astype(jnp.int32)
  hi = lax.bitcast_convert_type(d16[:, DW:], jnp.uint16).astype(jnp.int32)
  return lax.bitwise_or(lo, lax.shift_left(hi, 16))


def _mm0_body(x_ref, w_ref, o_ref):
  o_ref[...] = _pack_words(jnp.dot(
      x_ref[...], w_ref[...], preferred_element_type=jnp.float32))


def _mm0(x, w):
  grid = x.shape[0] // _BM
  return pl.pallas_call(
      _mm0_body,
      grid=(grid,),
      in_specs=[
          pl.BlockSpec((_BM, D), lambda i: (i, 0)),
          pl.BlockSpec((D, D), lambda i: (0, 0)),
      ],
      out_specs=pl.BlockSpec((_BM, DW), lambda i: (i, 0)),
      out_shape=jax.ShapeDtypeStruct((x.shape[0], DW), jnp.int32),
  )(x, w)


def _mm_mid_body(p0_ref, p1_ref, b_ref, w_ref, o_ref):
  h = jnp.maximum(p0_ref[...] + p1_ref[...] + b_ref[...], 0.0)
  o_ref[...] = _pack_words(jnp.dot(
      h, w_ref[...], preferred_element_type=jnp.float32))


def _mm_mid(p0, p1, b, w):
  grid = p0.shape[0] // _BM
  return pl.pallas_call(
      _mm_mid_body,
      grid=(grid,),
      in_specs=[
          pl.BlockSpec((_BM, D), lambda i: (i, 0)),
          pl.BlockSpec((_BM, D), lambda i: (i, 0)),
          pl.BlockSpec((1, D), lambda i: (0, 0)),
          pl.BlockSpec((D, D), lambda i: (0, 0)),
      ],
      out_specs=pl.BlockSpec((_BM, DW), lambda i: (i, 0)),
      out_shape=jax.ShapeDtypeStruct((p0.shape[0], DW), jnp.int32),
  )(p0, p1, b, w)


_BF = 1000


def _fin_body(p0_ref, p1_ref, b_ref, o_ref):
  h = p0_ref[...] + p1_ref[...] + b_ref[...]
  nrm = jnp.sqrt(jnp.sum(h * h, axis=1, keepdims=True))
  o_ref[...] = h / jnp.maximum(nrm, 1e-12)


def _fin(p0, p1, b):
  return pl.pallas_call(
      _fin_body,
      grid=(N // _BF,),
      in_specs=[
          pl.BlockSpec((_BF, D), lambda i: (i, 0)),
          pl.BlockSpec((_BF, D), lambda i: (i, 0)),
          pl.BlockSpec((1, D), lambda i: (0, 0)),
      ],
      out_specs=pl.BlockSpec((_BF, D), lambda i: (i, 0)),
      out_shape=jax.ShapeDtypeStruct((N, D), jnp.float32),
  )(p0, p1, b)


def kernel(x, adj, W1, b1, W2, b2, W3, b3):
  e = adj.shape[1]
  gran = NW * CHUNK * _K * _SEGS
  e_pad = ((e + gran - 1) // gran) * gran
  pad = e_pad - e

  src = jnp.concatenate([adj[0], jnp.zeros((pad,), jnp.int32)])
  dst = jnp.concatenate(
      [adj[1], N + (jnp.arange(pad, dtype=jnp.int32) % (N_PAD - N))]
  )
  xp = jnp.concatenate([x, jnp.zeros((N_PAD - N, D), jnp.float32)])
  zeros = jnp.zeros((ROWS_PER_TILE, D), jnp.float32)

  scatter = _make_scatter(e_pad)

  g = _mm0(xp, W1)
  p = scatter(g, src, dst, zeros)
  g = _mm_mid(p[0], p[1], b1.reshape(1, D), W2)
  p = scatter(g, src, dst, zeros)
  g = _mm_mid(p[0], p[1], b2.reshape(1, D), W3)
  p = scatter(g, src, dst, zeros)
  return _fin(p[0], p[1], b3.reshape(1, D))
